# baseline (speedup 1.0000x reference)
import functools

import jax
import jax.numpy as jnp
from jax import lax
from jax.experimental import pallas as pl
from jax.experimental.pallas import tpu as pltpu
from jax.experimental.pallas import tpu_sc as plsc

_D = 1024
_B = 4 * 8192
_NC = 2
_NS = 16
_NW = _NC * _NS
_BPW = _B // _NW
_CH = 8
_NCH = _BPW // _CH
_NBUF = 4


def _make_gather():
    mesh = plsc.VectorSubcoreMesh(core_axis_name="c", subcore_axis_name="s")

    @functools.partial(
        pl.kernel,
        mesh=mesh,
        out_type=jax.ShapeDtypeStruct((_B, _D), jnp.float32),
        scratch_types=[
            pltpu.VMEM((_NCH, _CH), jnp.int32),
            pltpu.VMEM_SHARED((_NS, _NBUF, _CH, _D), jnp.float32),
            *([pltpu.VMEM((_CH, _D), jnp.float32)] * _NBUF),
            *([pltpu.SemaphoreType.DMA] * (3 * _NBUF)),
        ],
    )
    def gather(idx_hbm, table_hbm, out_hbm, idx_v, shared, *bufs_and_sems):
        bufs = bufs_and_sems[:_NBUF]
        gsem = bufs_and_sems[_NBUF:2 * _NBUF]
        csem = bufs_and_sems[2 * _NBUF:3 * _NBUF]
        dsem = bufs_and_sems[3 * _NBUF:]
        sid = lax.axis_index("s")
        wid = sid * _NC + lax.axis_index("c")
        base = wid * _BPW

        def gather_chunk(c, b):
            return pltpu.make_async_copy(
                table_hbm.at[idx_v.at[c]], bufs[b], gsem[b])

        def cross_chunk(b):
            return pltpu.make_async_copy(
                bufs[b], shared.at[sid, b], csem[b])

        def drain_chunk(c, b):
            return pltpu.make_async_copy(
                shared.at[sid, b],
                out_hbm.at[pl.ds(base + c * _CH, _CH)], dsem[b])

        pltpu.sync_copy(idx_hbm.at[wid], idx_v)
        gather_chunk(0, 0).start()
        gather_chunk(1, 1).start()

        def body(g, carry):
            for b in range(_NBUF):
                c = _NBUF * g + b
                bn = (b + 2) % _NBUF
                n = c + 2
                gather_chunk(c, b).wait()

                @pl.when(c >= _NBUF)
                def _():
                    drain_chunk(c - _NBUF, b).wait()

                cross_chunk(b).start()

                @pl.when(c >= 2)
                def _():
                    cross_chunk(bn).wait()
                    drain_chunk(c - 2, bn).start()

                @pl.when(n < _NCH)
                def _():
                    gather_chunk(n, bn).start()
            return carry

        lax.fori_loop(0, _NCH // _NBUF, body, 0)
        for c in (_NCH - 2, _NCH - 1):
            cross_chunk(c % _NBUF).wait()
            drain_chunk(c, c % _NBUF).start()
        for c in range(_NCH - _NBUF, _NCH):
            drain_chunk(c, c % _NBUF).wait()

    return gather


_gather = _make_gather()


@jax.jit
def kernel(inputs, table):
    idx = inputs.reshape(_NW, _NCH, _CH)
    out = _gather(idx, table)
    return out.reshape(inputs.shape + (_D,))
